# baseline (speedup 1.0000x reference)
import functools

import jax
import jax.numpy as jnp
from jax import lax
from jax.experimental import pallas as pl
from jax.experimental.pallas import tpu as pltpu
from jax.experimental.pallas import tpu_sc as plsc

Q = 1024
K = 131072
D = 128
TOP_K = 3
BK = 2048
PAD = 8


def _topk_body(tc_ref, wc_ref, bc_ref, mk_ref, vals_ref, idx_ref, ck_ref):
    k = pl.program_id(0)

    @pl.when(k == 0)
    def _init():
        ck = lax.dot_general(tc_ref[...], wc_ref[...], (((1,), (1,)), ((), ())),
                             preferred_element_type=jnp.float32)
        ck_ref[...] = (ck + bc_ref[...]).astype(jnp.bfloat16)
        vals_ref[...] = jnp.full((Q, PAD), -jnp.inf, jnp.float32)
        idx_ref[...] = jnp.zeros((Q, PAD), jnp.int32)

    mk = mk_ref[...].astype(jnp.bfloat16)
    s = lax.dot_general(ck_ref[...], mk, (((1,), (1,)), ((), ())),
                        preferred_element_type=jnp.float32)

    bbits = lax.bitcast_convert_type(s, jnp.int32)
    k1 = bbits ^ ((bbits >> 31) & jnp.int32(0x7FFFFFFF))
    iota = lax.broadcasted_iota(jnp.int32, (Q, BK), 1)
    kk = (k1 | jnp.int32(2047)) - iota

    nchunks = BK // 128
    int_min = jnp.int32(-2147483648)
    m1 = kk[:, 0:128]
    m2 = jnp.full((Q, 128), int_min, jnp.int32)
    m3 = m2
    for c in range(1, nchunks):
        x = kk[:, c * 128:(c + 1) * 128]
        lo = jnp.minimum(m1, x)
        m1 = jnp.maximum(m1, x)
        lo2 = jnp.minimum(m2, lo)
        m2 = jnp.maximum(m2, lo)
        m3 = jnp.maximum(m3, lo2)

    cur, nxt, nxt2 = m1, m2, m3
    packed = []
    for t in range(TOP_K):
        mx = jnp.max(cur, axis=1, keepdims=True)
        packed.append(mx)
        if t < TOP_K - 1:
            hit = cur == mx
            cur = jnp.where(hit, nxt, cur)
            nxt = jnp.where(hit, nxt2, nxt)
            nxt2 = jnp.where(hit, int_min, nxt2)

    bv, bi = [], []
    for mx in packed:
        local = jnp.int32(2047) - (mx & jnp.int32(2047))
        bi.append(local + k * BK)
        kd = mx & jnp.int32(-2048)
        vb = kd ^ ((kd >> 31) & jnp.int32(0x7FFFFFFF))
        bv.append(lax.bitcast_convert_type(vb, jnp.float32))

    rv, ri = vals_ref[...], idx_ref[...]
    a1v, a2v, a3v = rv[:, 0:1], rv[:, 1:2], rv[:, 2:3]
    a1i, a2i, a3i = ri[:, 0:1], ri[:, 1:2], ri[:, 2:3]
    b1v, b2v, b3v = bv
    b1i, b2i, b3i = bi

    g1 = a1v >= b1v
    o1v = jnp.where(g1, a1v, b1v)
    o1i = jnp.where(g1, a1i, b1i)
    pav = jnp.where(g1, a2v, a1v)
    pai = jnp.where(g1, a2i, a1i)
    pav2 = jnp.where(g1, a3v, a2v)
    pai2 = jnp.where(g1, a3i, a2i)
    pbv = jnp.where(g1, b1v, b2v)
    pbi = jnp.where(g1, b1i, b2i)
    pbv2 = jnp.where(g1, b2v, b3v)
    pbi2 = jnp.where(g1, b2i, b3i)

    g2 = pav >= pbv
    o2v = jnp.where(g2, pav, pbv)
    o2i = jnp.where(g2, pai, pbi)
    qav = jnp.where(g2, pav2, pav)
    qai = jnp.where(g2, pai2, pai)
    qbv = jnp.where(g2, pbv, pbv2)
    qbi = jnp.where(g2, pbi, pbi2)

    g3 = qav >= qbv
    o3v = jnp.where(g3, qav, qbv)
    o3i = jnp.where(g3, qai, qbi)

    pad_v = jnp.full((Q, PAD - TOP_K), -jnp.inf, jnp.float32)
    pad_i = jnp.zeros((Q, PAD - TOP_K), jnp.int32)
    vals_ref[...] = jnp.concatenate([o1v, o2v, o3v, pad_v], axis=1)
    idx_ref[...] = jnp.concatenate([o1i, o2i, o3i, pad_i], axis=1)


def _topk_call(token_content, W_content, b_content_row, mem_keys):
    return pl.pallas_call(
        _topk_body,
        grid=(K // BK,),
        in_specs=[
            pl.BlockSpec((Q, D), lambda k: (0, 0)),
            pl.BlockSpec((D, D), lambda k: (0, 0)),
            pl.BlockSpec((1, D), lambda k: (0, 0)),
            pl.BlockSpec((BK, D), lambda k: (k, 0)),
        ],
        out_specs=[
            pl.BlockSpec((Q, PAD), lambda k: (0, 0)),
            pl.BlockSpec((Q, PAD), lambda k: (0, 0)),
        ],
        out_shape=[
            jax.ShapeDtypeStruct((Q, PAD), jnp.float32),
            jax.ShapeDtypeStruct((Q, PAD), jnp.int32),
        ],
        scratch_shapes=[pltpu.VMEM((Q, D), jnp.bfloat16)],
        compiler_params=pltpu.CompilerParams(
            dimension_semantics=("arbitrary",)),
    )(token_content, W_content, b_content_row, mem_keys)


def _gather_call(flat_idx, table):
    B = flat_idx.shape[0]
    info = plsc.get_sparse_core_info()
    nc, ns = info.num_cores, info.num_subcores
    nw = nc * ns
    b_per_w = B // nw
    mesh = plsc.VectorSubcoreMesh(core_axis_name="c", subcore_axis_name="s")

    @functools.partial(
        pl.kernel, mesh=mesh,
        out_type=jax.ShapeDtypeStruct((B, D), jnp.float32),
        scratch_types=[
            pltpu.VMEM((b_per_w,), jnp.int32),
            pltpu.VMEM((b_per_w, D), jnp.float32),
            pltpu.SemaphoreType.DMA,
        ],
    )
    def gather_k(idx_hbm, table_hbm, out_hbm, idx_v, rows_v, sem):
        wid = lax.axis_index("s") * nc + lax.axis_index("c")
        base = wid * b_per_w
        pltpu.sync_copy(idx_hbm.at[pl.ds(base, b_per_w)], idx_v)
        pltpu.async_copy(table_hbm.at[idx_v], rows_v, sem).wait()
        pltpu.sync_copy(rows_v, out_hbm.at[pl.ds(base, b_per_w)])

    return gather_k(flat_idx, table)


def _epilogue_body(posf_ref, wpos_ref, bpos_ref, vals_ref, gath_ref, ts_ref,
                   sw_ref, wg_ref, bg_ref, we_ref, be_ref, out_ref):
    v = vals_ref[...]
    m = v[:, 0:1]
    e = jnp.exp(v - m)
    attn = e / jnp.sum(e, axis=1, keepdims=True)
    g = gath_ref[...]
    sim = (attn[:, 0:1] * g[:, 0:D]
           + attn[:, 1:2] * g[:, D:2 * D]
           + attn[:, 2:3] * g[:, 2 * D:3 * D])
    base = posf_ref[...] * wpos_ref[...] + bpos_ref[...]
    fe = base + sw_ref[...] * sim
    gate_in = jnp.concatenate([fe, ts_ref[...]], axis=1)
    z = lax.dot_general(gate_in, wg_ref[...], (((1,), (1,)), ((), ())),
                        preferred_element_type=jnp.float32) + bg_ref[...]
    ti = jax.nn.sigmoid(z)
    ev = lax.dot_general(fe, we_ref[...], (((1,), (1,)), ((), ())),
                         preferred_element_type=jnp.float32) + be_ref[...]
    out_ref[...] = fe + ti * ev


def _epilogue_call(pos_f, wpos_row, bpos_row, top_vals, gathered,
                   temporal_state, sw, W_gate, bg_row, W_evol, be_row):
    return pl.pallas_call(
        _epilogue_body,
        out_shape=jax.ShapeDtypeStruct((Q, D), jnp.float32),
    )(pos_f, wpos_row, bpos_row, top_vals, gathered, temporal_state,
      sw, W_gate, bg_row, W_evol, be_row)


def kernel(positions, token_content, temporal_state, mem_keys, mem_values,
           W_pos, b_pos, W_content, b_content, similarity_weight,
           W_gate, b_gate, W_evol, b_evol):
    top_vals, top_idx = _topk_call(
        token_content, W_content, b_content.reshape(1, D), mem_keys)

    flat_idx = top_idx[:, :TOP_K].reshape(-1)
    gathered = _gather_call(flat_idx, mem_values)
    gathered = gathered.reshape(Q, TOP_K * D)

    pos_f = positions.astype(jnp.float32).reshape(Q, 1)
    return _epilogue_call(
        pos_f,
        W_pos.reshape(1, D),
        b_pos.reshape(1, D),
        top_vals,
        gathered,
        temporal_state,
        similarity_weight.reshape(1, 1),
        W_gate,
        b_gate.reshape(1, D),
        W_evol,
        b_evol.reshape(1, D),
    )


# speedup vs baseline: 1.4605x; 1.4605x over previous
import functools

import jax
import jax.numpy as jnp
from jax import lax
from jax.experimental import pallas as pl
from jax.experimental.pallas import tpu as pltpu
from jax.experimental.pallas import tpu_sc as plsc

Q = 1024
K = 131072
D = 128
TOP_K = 3
BK = 2048
PAD = 8


def _topk_body(tc_ref, wc_ref, bc_ref, mk_ref, vals_ref, idx_ref, ck_ref):
    k = pl.program_id(0)

    @pl.when(k == 0)
    def _init():
        ck = lax.dot_general(tc_ref[...], wc_ref[...], (((1,), (1,)), ((), ())),
                             preferred_element_type=jnp.float32)
        ck_ref[...] = (ck + bc_ref[...]).astype(jnp.bfloat16)
        vals_ref[...] = jnp.full((Q, PAD), -jnp.inf, jnp.float32)
        idx_ref[...] = jnp.zeros((Q, PAD), jnp.int32)

    mk = mk_ref[...].astype(jnp.bfloat16)
    s = lax.dot_general(ck_ref[...], mk, (((1,), (1,)), ((), ())),
                        preferred_element_type=jnp.float32)

    bbits = lax.bitcast_convert_type(s, jnp.int32)
    u = bbits ^ ((bbits >> 31) | jnp.int32(-2147483648))
    iota = lax.broadcasted_iota(jnp.int32, (Q, BK), 1)
    ki = (lax.shift_right_logical(u, 2) & jnp.int32(0x3FFFF800)) | iota
    kk = lax.bitcast_convert_type(ki, jnp.float32)

    nchunks = BK // 128
    m1 = kk[:, 0:128]
    m2 = jnp.zeros((Q, 128), jnp.float32) - 1.0
    m3 = m2
    for c in range(1, nchunks):
        x = kk[:, c * 128:(c + 1) * 128]
        lo = jnp.minimum(m1, x)
        m1 = jnp.maximum(m1, x)
        lo2 = jnp.minimum(m2, lo)
        m2 = jnp.maximum(m2, lo)
        m3 = jnp.maximum(m3, lo2)

    cur, nxt, nxt2 = m1, m2, m3
    packed = []
    for t in range(TOP_K):
        mx = jnp.max(cur, axis=1, keepdims=True)
        packed.append(mx)
        if t < TOP_K - 1:
            hit = cur == mx
            cur = jnp.where(hit, nxt, cur)
            nxt = jnp.where(hit, nxt2, nxt)
            nxt2 = jnp.where(hit, jnp.float32(-1.0), nxt2)

    bv, bi = [], []
    for mx in packed:
        mi = lax.bitcast_convert_type(mx, jnp.int32)
        bi.append((mi & jnp.int32(2047)) + k * BK)
        ut = lax.shift_left(mi & jnp.int32(0x3FFFF800), 2)
        vb = ut ^ (((~ut) >> 31) | jnp.int32(-2147483648))
        bv.append(lax.bitcast_convert_type(vb, jnp.float32))

    rv, ri = vals_ref[...], idx_ref[...]
    a1v, a2v, a3v = rv[:, 0:1], rv[:, 1:2], rv[:, 2:3]
    a1i, a2i, a3i = ri[:, 0:1], ri[:, 1:2], ri[:, 2:3]
    b1v, b2v, b3v = bv
    b1i, b2i, b3i = bi

    g1 = a1v >= b1v
    o1v = jnp.where(g1, a1v, b1v)
    o1i = jnp.where(g1, a1i, b1i)
    pav = jnp.where(g1, a2v, a1v)
    pai = jnp.where(g1, a2i, a1i)
    pav2 = jnp.where(g1, a3v, a2v)
    pai2 = jnp.where(g1, a3i, a2i)
    pbv = jnp.where(g1, b1v, b2v)
    pbi = jnp.where(g1, b1i, b2i)
    pbv2 = jnp.where(g1, b2v, b3v)
    pbi2 = jnp.where(g1, b2i, b3i)

    g2 = pav >= pbv
    o2v = jnp.where(g2, pav, pbv)
    o2i = jnp.where(g2, pai, pbi)
    qav = jnp.where(g2, pav2, pav)
    qai = jnp.where(g2, pai2, pai)
    qbv = jnp.where(g2, pbv, pbv2)
    qbi = jnp.where(g2, pbi, pbi2)

    g3 = qav >= qbv
    o3v = jnp.where(g3, qav, qbv)
    o3i = jnp.where(g3, qai, qbi)

    pad_v = jnp.full((Q, PAD - TOP_K), -jnp.inf, jnp.float32)
    pad_i = jnp.zeros((Q, PAD - TOP_K), jnp.int32)
    vals_ref[...] = jnp.concatenate([o1v, o2v, o3v, pad_v], axis=1)
    idx_ref[...] = jnp.concatenate([o1i, o2i, o3i, pad_i], axis=1)


def _topk_call(token_content, W_content, b_content_row, mem_keys):
    return pl.pallas_call(
        _topk_body,
        grid=(K // BK,),
        in_specs=[
            pl.BlockSpec((Q, D), lambda k: (0, 0)),
            pl.BlockSpec((D, D), lambda k: (0, 0)),
            pl.BlockSpec((1, D), lambda k: (0, 0)),
            pl.BlockSpec((BK, D), lambda k: (k, 0)),
        ],
        out_specs=[
            pl.BlockSpec((Q, PAD), lambda k: (0, 0)),
            pl.BlockSpec((Q, PAD), lambda k: (0, 0)),
        ],
        out_shape=[
            jax.ShapeDtypeStruct((Q, PAD), jnp.float32),
            jax.ShapeDtypeStruct((Q, PAD), jnp.int32),
        ],
        scratch_shapes=[pltpu.VMEM((Q, D), jnp.bfloat16)],
        compiler_params=pltpu.CompilerParams(
            dimension_semantics=("arbitrary",)),
    )(token_content, W_content, b_content_row, mem_keys)


def _gather_call(flat_idx, table):
    B = flat_idx.shape[0]
    info = plsc.get_sparse_core_info()
    nc, ns = info.num_cores, info.num_subcores
    nw = nc * ns
    b_per_w = B // nw
    mesh = plsc.VectorSubcoreMesh(core_axis_name="c", subcore_axis_name="s")

    @functools.partial(
        pl.kernel, mesh=mesh,
        out_type=jax.ShapeDtypeStruct((B, D), jnp.float32),
        scratch_types=[
            pltpu.VMEM((b_per_w,), jnp.int32),
            pltpu.VMEM((b_per_w, D), jnp.float32),
            pltpu.SemaphoreType.DMA,
        ],
    )
    def gather_k(idx_hbm, table_hbm, out_hbm, idx_v, rows_v, sem):
        wid = lax.axis_index("s") * nc + lax.axis_index("c")
        base = wid * b_per_w
        pltpu.sync_copy(idx_hbm.at[pl.ds(base, b_per_w)], idx_v)
        pltpu.async_copy(table_hbm.at[idx_v], rows_v, sem).wait()
        pltpu.sync_copy(rows_v, out_hbm.at[pl.ds(base, b_per_w)])

    return gather_k(flat_idx, table)


def _epilogue_body(posf_ref, wpos_ref, bpos_ref, vals_ref, gath_ref, ts_ref,
                   sw_ref, wg_ref, bg_ref, we_ref, be_ref, out_ref):
    v = vals_ref[...]
    m = v[:, 0:1]
    e = jnp.exp(v - m)
    attn = e / jnp.sum(e, axis=1, keepdims=True)
    g = gath_ref[...]
    sim = (attn[:, 0:1] * g[:, 0:D]
           + attn[:, 1:2] * g[:, D:2 * D]
           + attn[:, 2:3] * g[:, 2 * D:3 * D])
    base = posf_ref[...] * wpos_ref[...] + bpos_ref[...]
    fe = base + sw_ref[...] * sim
    gate_in = jnp.concatenate([fe, ts_ref[...]], axis=1)
    z = lax.dot_general(gate_in, wg_ref[...], (((1,), (1,)), ((), ())),
                        preferred_element_type=jnp.float32) + bg_ref[...]
    ti = jax.nn.sigmoid(z)
    ev = lax.dot_general(fe, we_ref[...], (((1,), (1,)), ((), ())),
                         preferred_element_type=jnp.float32) + be_ref[...]
    out_ref[...] = fe + ti * ev


def _epilogue_call(pos_f, wpos_row, bpos_row, top_vals, gathered,
                   temporal_state, sw, W_gate, bg_row, W_evol, be_row):
    return pl.pallas_call(
        _epilogue_body,
        out_shape=jax.ShapeDtypeStruct((Q, D), jnp.float32),
    )(pos_f, wpos_row, bpos_row, top_vals, gathered, temporal_state,
      sw, W_gate, bg_row, W_evol, be_row)


def kernel(positions, token_content, temporal_state, mem_keys, mem_values,
           W_pos, b_pos, W_content, b_content, similarity_weight,
           W_gate, b_gate, W_evol, b_evol):
    top_vals, top_idx = _topk_call(
        token_content, W_content, b_content.reshape(1, D), mem_keys)

    flat_idx = top_idx[:, :TOP_K].reshape(-1)
    gathered = _gather_call(flat_idx, mem_values)
    gathered = gathered.reshape(Q, TOP_K * D)

    pos_f = positions.astype(jnp.float32).reshape(Q, 1)
    return _epilogue_call(
        pos_f,
        W_pos.reshape(1, D),
        b_pos.reshape(1, D),
        top_vals,
        gathered,
        temporal_state,
        similarity_weight.reshape(1, 1),
        W_gate,
        b_gate.reshape(1, D),
        W_evol,
        b_evol.reshape(1, D),
    )


# speedup vs baseline: 2.9342x; 2.0090x over previous
import functools

import jax
import jax.numpy as jnp
from jax import lax
from jax.experimental import pallas as pl
from jax.experimental.pallas import tpu as pltpu
from jax.experimental.pallas import tpu_sc as plsc

Q = 1024
K = 131072
D = 128
TOP_K = 3
BK = 2048
PAD = 8

_VMASK = -131072
_IMASK = 131071


def _topk_body(tc_ref, wc_ref, bc_ref, mk_ref, idx_ref, ck_ref, ckb_ref,
               m1_ref, m2_ref, m3_ref):
    k = pl.program_id(0)

    @pl.when(k == 0)
    def _init():
        ck = lax.dot_general(tc_ref[...], wc_ref[...], (((1,), (1,)), ((), ())),
                             preferred_element_type=jnp.float32)
        ck = ck + bc_ref[...]
        ck_ref[...] = ck
        ckb_ref[...] = ck.astype(jnp.bfloat16)
        m1_ref[...] = jnp.full((Q, 128), -jnp.inf, jnp.float32)
        m2_ref[...] = jnp.full((Q, 128), -jnp.inf, jnp.float32)
        m3_ref[...] = jnp.full((Q, 128), -jnp.inf, jnp.float32)

    mk = mk_ref[...].astype(jnp.bfloat16)
    s = lax.dot_general(ckb_ref[...], mk, (((1,), (1,)), ((), ())),
                        preferred_element_type=jnp.float32)

    bbits = lax.bitcast_convert_type(s, jnp.int32)
    gidx = lax.broadcasted_iota(jnp.int32, (Q, BK), 1) + k * BK
    kf = lax.bitcast_convert_type((bbits & jnp.int32(_VMASK)) | gidx, jnp.float32)

    m1, m2, m3 = m1_ref[...], m2_ref[...], m3_ref[...]
    for c in range(BK // 128):
        x = kf[:, c * 128:(c + 1) * 128]
        lo = jnp.minimum(m1, x)
        m1 = jnp.maximum(m1, x)
        lo2 = jnp.minimum(m2, lo)
        m2 = jnp.maximum(m2, lo)
        m3 = jnp.maximum(m3, lo2)
    m1_ref[...] = m1
    m2_ref[...] = m2
    m3_ref[...] = m3

    @pl.when(k == pl.num_programs(0) - 1)
    def _extract():
        cur, nxt, nxt2 = m1, m2, m3
        out = []
        for t in range(TOP_K):
            mx = jnp.max(cur, axis=1, keepdims=True)
            mi = lax.bitcast_convert_type(mx, jnp.int32)
            out.append(mi & jnp.int32(_IMASK))
            if t < TOP_K - 1:
                hit = cur == mx
                cur = jnp.where(hit, nxt, cur)
                nxt = jnp.where(hit, nxt2, nxt)
                nxt2 = jnp.where(hit, jnp.float32(-jnp.inf), nxt2)
        pad_i = jnp.zeros((Q, PAD - TOP_K), jnp.int32)
        idx_ref[...] = jnp.concatenate(out + [pad_i], axis=1)


def _topk_call(token_content, W_content, b_content_row, mem_keys):
    return pl.pallas_call(
        _topk_body,
        grid=(K // BK,),
        in_specs=[
            pl.BlockSpec((Q, D), lambda k: (0, 0)),
            pl.BlockSpec((D, D), lambda k: (0, 0)),
            pl.BlockSpec((1, D), lambda k: (0, 0)),
            pl.BlockSpec((BK, D), lambda k: (k, 0)),
        ],
        out_specs=[
            pl.BlockSpec((Q, PAD), lambda k: (0, 0)),
            pl.BlockSpec((Q, D), lambda k: (0, 0)),
        ],
        out_shape=[
            jax.ShapeDtypeStruct((Q, PAD), jnp.int32),
            jax.ShapeDtypeStruct((Q, D), jnp.float32),
        ],
        scratch_shapes=[
            pltpu.VMEM((Q, D), jnp.bfloat16),
            pltpu.VMEM((Q, 128), jnp.float32),
            pltpu.VMEM((Q, 128), jnp.float32),
            pltpu.VMEM((Q, 128), jnp.float32),
        ],
        compiler_params=pltpu.CompilerParams(
            dimension_semantics=("arbitrary",)),
    )(token_content, W_content, b_content_row, mem_keys)


def _gather_call(flat_idx, values, keys):
    B = flat_idx.shape[0]
    info = plsc.get_sparse_core_info()
    nc, ns = info.num_cores, info.num_subcores
    nw = nc * ns
    b_per_w = B // nw
    mesh = plsc.VectorSubcoreMesh(core_axis_name="c", subcore_axis_name="s")

    @functools.partial(
        pl.kernel, mesh=mesh,
        out_type=[
            jax.ShapeDtypeStruct((B, D), jnp.float32),
            jax.ShapeDtypeStruct((B, D), jnp.float32),
        ],
        scratch_types=[
            pltpu.VMEM((b_per_w,), jnp.int32),
            pltpu.VMEM((b_per_w, D), jnp.float32),
            pltpu.VMEM((b_per_w, D), jnp.float32),
            pltpu.SemaphoreType.DMA,
            pltpu.SemaphoreType.DMA,
        ],
    )
    def gather_k(idx_hbm, val_hbm, key_hbm, outv_hbm, outk_hbm,
                 idx_v, rows_v, rows_k, semv, semk):
        wid = lax.axis_index("s") * nc + lax.axis_index("c")
        base = wid * b_per_w
        pltpu.sync_copy(idx_hbm.at[pl.ds(base, b_per_w)], idx_v)
        cpv = pltpu.async_copy(val_hbm.at[idx_v], rows_v, semv)
        cpk = pltpu.async_copy(key_hbm.at[idx_v], rows_k, semk)
        cpv.wait()
        cpk.wait()
        pltpu.sync_copy(rows_v, outv_hbm.at[pl.ds(base, b_per_w)])
        pltpu.sync_copy(rows_k, outk_hbm.at[pl.ds(base, b_per_w)])

    return gather_k(flat_idx, values, keys)


def _epilogue_body(posf_ref, wpos_ref, bpos_ref, ck_ref, gv_ref, gk_ref,
                   ts_ref, sw_ref, wg_ref, bg_ref, we_ref, be_ref, out_ref):
    ck = ck_ref[...]
    sims = []
    for t in range(TOP_K):
        gk = gk_ref[:, t * D:(t + 1) * D]
        sims.append(jnp.sum(ck * gk, axis=1, keepdims=True))
    m = jnp.maximum(jnp.maximum(sims[0], sims[1]), sims[2])
    e = [jnp.exp(sv - m) for sv in sims]
    denom = e[0] + e[1] + e[2]
    sim = e[0] * gv_ref[:, 0:D] + e[1] * gv_ref[:, D:2 * D] \
        + e[2] * gv_ref[:, 2 * D:3 * D]
    sim = sim / denom
    base = posf_ref[...] * wpos_ref[...] + bpos_ref[...]
    fe = base + sw_ref[...] * sim
    gate_in = jnp.concatenate([fe, ts_ref[...]], axis=1)
    z = lax.dot_general(gate_in, wg_ref[...], (((1,), (1,)), ((), ())),
                        preferred_element_type=jnp.float32) + bg_ref[...]
    ti = jax.nn.sigmoid(z)
    ev = lax.dot_general(fe, we_ref[...], (((1,), (1,)), ((), ())),
                         preferred_element_type=jnp.float32) + be_ref[...]
    out_ref[...] = fe + ti * ev


def _epilogue_call(pos_f, wpos_row, bpos_row, ck, gath_v, gath_k,
                   temporal_state, sw, W_gate, bg_row, W_evol, be_row):
    return pl.pallas_call(
        _epilogue_body,
        out_shape=jax.ShapeDtypeStruct((Q, D), jnp.float32),
    )(pos_f, wpos_row, bpos_row, ck, gath_v, gath_k, temporal_state,
      sw, W_gate, bg_row, W_evol, be_row)


def kernel(positions, token_content, temporal_state, mem_keys, mem_values,
           W_pos, b_pos, W_content, b_content, similarity_weight,
           W_gate, b_gate, W_evol, b_evol):
    top_idx, ck = _topk_call(
        token_content, W_content, b_content.reshape(1, D), mem_keys)

    flat_idx = top_idx[:, :TOP_K].reshape(-1)
    gath_v, gath_k = _gather_call(flat_idx, mem_values, mem_keys)
    gath_v = gath_v.reshape(Q, TOP_K * D)
    gath_k = gath_k.reshape(Q, TOP_K * D)

    pos_f = positions.astype(jnp.float32).reshape(Q, 1)
    return _epilogue_call(
        pos_f,
        W_pos.reshape(1, D),
        b_pos.reshape(1, D),
        ck,
        gath_v,
        gath_k,
        temporal_state,
        similarity_weight.reshape(1, 1),
        W_gate,
        b_gate.reshape(1, D),
        W_evol,
        b_evol.reshape(1, D),
    )


# speedup vs baseline: 3.3755x; 1.1504x over previous
import functools

import jax
import jax.numpy as jnp
from jax import lax
from jax.experimental import pallas as pl
from jax.experimental.pallas import tpu as pltpu
from jax.experimental.pallas import tpu_sc as plsc

Q = 1024
K = 131072
D = 128
TOP_K = 3
BK = 2048
PAD = 8

_VMASK = -131072
_IMASK = 131071


def _topk_body(tc_ref, wc_ref, bc_ref, mk_ref, idx_ref, ck_ref, ckb_ref,
               m1_ref, m2_ref, m3_ref):
    k = pl.program_id(0)

    @pl.when(k == 0)
    def _init():
        ck = lax.dot_general(tc_ref[...], wc_ref[...], (((1,), (1,)), ((), ())),
                             preferred_element_type=jnp.float32)
        ck = ck + bc_ref[...]
        ck_ref[...] = ck
        ckb_ref[...] = ck.astype(jnp.bfloat16)
        m1_ref[...] = jnp.full((Q, 128), -jnp.inf, jnp.float32)
        m2_ref[...] = jnp.full((Q, 128), -jnp.inf, jnp.float32)
        m3_ref[...] = jnp.full((Q, 128), -jnp.inf, jnp.float32)

    mk = mk_ref[...].astype(jnp.bfloat16)
    s = lax.dot_general(ckb_ref[...], mk, (((1,), (1,)), ((), ())),
                        preferred_element_type=jnp.float32)

    g = lax.broadcasted_iota(jnp.int32, (1, BK), 1) + k * BK

    def key_chunk(c):
        xb = lax.bitcast_convert_type(s[:, c * 128:(c + 1) * 128], jnp.int32)
        ki = (xb & jnp.int32(_VMASK)) | g[:, c * 128:(c + 1) * 128]
        return lax.bitcast_convert_type(ki, jnp.float32)

    m1, m2, m3 = m1_ref[...], m2_ref[...], m3_ref[...]
    for c in range(0, BK // 128, 2):
        x1 = key_chunk(c)
        x2 = key_chunk(c + 1)
        hi = jnp.maximum(x1, x2)
        lo = jnp.minimum(x1, x2)
        u = jnp.minimum(m1, hi)
        m1 = jnp.maximum(m1, hi)
        q = jnp.maximum(m2, lo)
        v = jnp.minimum(u, q)
        m2 = jnp.maximum(u, q)
        m3 = jnp.maximum(v, m3)
    m1_ref[...] = m1
    m2_ref[...] = m2
    m3_ref[...] = m3

    @pl.when(k == pl.num_programs(0) - 1)
    def _extract():
        cur, nxt, nxt2 = m1, m2, m3
        out = []
        for t in range(TOP_K):
            mx = jnp.max(cur, axis=1, keepdims=True)
            mi = lax.bitcast_convert_type(mx, jnp.int32)
            out.append(mi & jnp.int32(_IMASK))
            if t < TOP_K - 1:
                hit = cur == mx
                cur = jnp.where(hit, nxt, cur)
                nxt = jnp.where(hit, nxt2, nxt)
                nxt2 = jnp.where(hit, jnp.float32(-jnp.inf), nxt2)
        pad_i = jnp.zeros((Q, PAD - TOP_K), jnp.int32)
        idx_ref[...] = jnp.concatenate(out + [pad_i], axis=1)


def _topk_call(token_content, W_content, b_content_row, mem_keys):
    return pl.pallas_call(
        _topk_body,
        grid=(K // BK,),
        in_specs=[
            pl.BlockSpec((Q, D), lambda k: (0, 0)),
            pl.BlockSpec((D, D), lambda k: (0, 0)),
            pl.BlockSpec((1, D), lambda k: (0, 0)),
            pl.BlockSpec((BK, D), lambda k: (k, 0)),
        ],
        out_specs=[
            pl.BlockSpec((Q, PAD), lambda k: (0, 0)),
            pl.BlockSpec((Q, D), lambda k: (0, 0)),
        ],
        out_shape=[
            jax.ShapeDtypeStruct((Q, PAD), jnp.int32),
            jax.ShapeDtypeStruct((Q, D), jnp.float32),
        ],
        scratch_shapes=[
            pltpu.VMEM((Q, D), jnp.bfloat16),
            pltpu.VMEM((Q, 128), jnp.float32),
            pltpu.VMEM((Q, 128), jnp.float32),
            pltpu.VMEM((Q, 128), jnp.float32),
        ],
        compiler_params=pltpu.CompilerParams(
            dimension_semantics=("arbitrary",)),
    )(token_content, W_content, b_content_row, mem_keys)


def _gather_call(flat_idx, values, keys):
    B = flat_idx.shape[0]
    info = plsc.get_sparse_core_info()
    nc, ns = info.num_cores, info.num_subcores
    nw = nc * ns
    b_per_w = B // nw
    mesh = plsc.VectorSubcoreMesh(core_axis_name="c", subcore_axis_name="s")

    @functools.partial(
        pl.kernel, mesh=mesh,
        out_type=[
            jax.ShapeDtypeStruct((B, D), jnp.float32),
            jax.ShapeDtypeStruct((B, D), jnp.float32),
        ],
        scratch_types=[
            pltpu.VMEM((b_per_w,), jnp.int32),
            pltpu.VMEM((b_per_w, D), jnp.float32),
            pltpu.VMEM((b_per_w, D), jnp.float32),
            pltpu.SemaphoreType.DMA,
            pltpu.SemaphoreType.DMA,
        ],
    )
    def gather_k(idx_hbm, val_hbm, key_hbm, outv_hbm, outk_hbm,
                 idx_v, rows_v, rows_k, semv, semk):
        wid = lax.axis_index("s") * nc + lax.axis_index("c")
        base = wid * b_per_w
        pltpu.sync_copy(idx_hbm.at[pl.ds(base, b_per_w)], idx_v)
        cpv = pltpu.async_copy(val_hbm.at[idx_v], rows_v, semv)
        cpk = pltpu.async_copy(key_hbm.at[idx_v], rows_k, semk)
        cpv.wait()
        cpk.wait()
        pltpu.sync_copy(rows_v, outv_hbm.at[pl.ds(base, b_per_w)])
        pltpu.sync_copy(rows_k, outk_hbm.at[pl.ds(base, b_per_w)])

    return gather_k(flat_idx, values, keys)


def _epilogue_body(posf_ref, wpos_ref, bpos_ref, ck_ref,
                   gv0_ref, gv1_ref, gv2_ref, gk0_ref, gk1_ref, gk2_ref,
                   ts_ref, sw_ref, wg_ref, bg_ref, we_ref, be_ref, out_ref):
    ck = ck_ref[...]
    sims = [jnp.sum(ck * gk_ref[...], axis=1, keepdims=True)
            for gk_ref in (gk0_ref, gk1_ref, gk2_ref)]
    m = jnp.maximum(jnp.maximum(sims[0], sims[1]), sims[2])
    e = [jnp.exp(sv - m) for sv in sims]
    denom = e[0] + e[1] + e[2]
    sim = e[0] * gv0_ref[...] + e[1] * gv1_ref[...] + e[2] * gv2_ref[...]
    sim = sim / denom
    base = posf_ref[...] * wpos_ref[...] + bpos_ref[...]
    fe = base + sw_ref[...] * sim
    gate_in = jnp.concatenate([fe, ts_ref[...]], axis=1)
    z = lax.dot_general(gate_in, wg_ref[...], (((1,), (1,)), ((), ())),
                        preferred_element_type=jnp.float32) + bg_ref[...]
    ti = jax.nn.sigmoid(z)
    ev = lax.dot_general(fe, we_ref[...], (((1,), (1,)), ((), ())),
                         preferred_element_type=jnp.float32) + be_ref[...]
    out_ref[...] = fe + ti * ev


def _epilogue_call(pos_f, wpos_row, bpos_row, ck, gvs, gks,
                   temporal_state, sw, W_gate, bg_row, W_evol, be_row):
    return pl.pallas_call(
        _epilogue_body,
        out_shape=jax.ShapeDtypeStruct((Q, D), jnp.float32),
    )(pos_f, wpos_row, bpos_row, ck, *gvs, *gks, temporal_state,
      sw, W_gate, bg_row, W_evol, be_row)


def kernel(positions, token_content, temporal_state, mem_keys, mem_values,
           W_pos, b_pos, W_content, b_content, similarity_weight,
           W_gate, b_gate, W_evol, b_evol):
    top_idx, ck = _topk_call(
        token_content, W_content, b_content.reshape(1, D), mem_keys)

    flat_idx = top_idx[:, :TOP_K].T.reshape(-1)
    gath_v, gath_k = _gather_call(flat_idx, mem_values, mem_keys)
    gvs = [gath_v[t * Q:(t + 1) * Q] for t in range(TOP_K)]
    gks = [gath_k[t * Q:(t + 1) * Q] for t in range(TOP_K)]

    pos_f = positions.astype(jnp.float32).reshape(Q, 1)
    return _epilogue_call(
        pos_f,
        W_pos.reshape(1, D),
        b_pos.reshape(1, D),
        ck,
        gvs,
        gks,
        temporal_state,
        similarity_weight.reshape(1, 1),
        W_gate,
        b_gate.reshape(1, D),
        W_evol,
        b_evol.reshape(1, D),
    )
